# Initial kernel scaffold; baseline (speedup 1.0000x reference)
import jax
import jax.numpy as jnp
from jax.experimental import pallas as pl


def kernel(x, edge_index, batch, W1, b1, W2, b2, W3, b3, W4, b4, Wfc, bfc):
    raise NotImplementedError("write your pallas kernel here")


# speedup vs baseline: 14.2140x; 14.2140x over previous
import functools

import jax
import jax.numpy as jnp
from jax import lax
from jax.experimental import pallas as pl
from jax.experimental.pallas import tpu as pltpu
from jax.experimental.pallas import tpu_sc as plsc

NN = 10000
EE = 320000
DH = 128
DOUT = 64
GG = 64
NC = 2
NS = 16
NW = NC * NS
EPW = EE // NW
CH = 80
NCH = EPW // CH
RA = 624
TAIL = NN - NS * RA
DEGW = 16

BR = 1000
NBLK = NN // BR

_MESH = plsc.VectorSubcoreMesh(
    core_axis_name="c", subcore_axis_name="s", num_cores=NC, num_subcores=NS)


def _deg_body(dstr, zer16, out, dst_v, buf, acc):
  c = lax.axis_index("c")
  s = lax.axis_index("s")
  wid = s * NC + c
  pltpu.sync_copy(zer16.at[pl.ds(s * RA, RA)], acc.at[pl.ds(s * RA, RA)])

  @pl.when(s == NS - 1)
  def _():
    pltpu.sync_copy(zer16.at[pl.ds(NS * RA, TAIL)],
                    acc.at[pl.ds(NS * RA, TAIL)])

  pltpu.sync_copy(dstr.at[wid], dst_v)

  def fill(i, _):
    buf[i] = jnp.ones((DEGW,), jnp.float32)
    return 0
  lax.fori_loop(0, CH, fill, 0)
  plsc.subcore_barrier()

  def chunk(j, _):
    pltpu.sync_copy(buf.at[pl.ds(0, CH)], acc.at[dst_v.at[j]], add=True)
    return 0
  lax.fori_loop(0, NCH, chunk, 0)
  plsc.subcore_barrier()
  pltpu.sync_copy(acc.at[pl.ds(s * RA, RA)], out.at[c, pl.ds(s * RA, RA)])

  @pl.when(s == NS - 1)
  def _():
    pltpu.sync_copy(acc.at[pl.ds(NS * RA, TAIL)],
                    out.at[c, pl.ds(NS * RA, TAIL)])


_deg_call = pl.kernel(
    _deg_body,
    out_type=jax.ShapeDtypeStruct((NC, NN, DEGW), jnp.float32),
    mesh=_MESH,
    scratch_types=[
        pltpu.VMEM((NCH, CH), jnp.int32),
        pltpu.VMEM((CH, DEGW), jnp.float32),
        pltpu.VMEM_SHARED((NN, DEGW), jnp.float32),
    ],
)


def _edge_body(p_hbm, srcr, dstr, zer, out, src_v, dst_v, rows, acc, sem):
  c = lax.axis_index("c")
  s = lax.axis_index("s")
  wid = s * NC + c
  pltpu.sync_copy(zer.at[pl.ds(s * RA, RA)], acc.at[pl.ds(s * RA, RA)])

  @pl.when(s == NS - 1)
  def _():
    pltpu.sync_copy(zer.at[pl.ds(NS * RA, TAIL)],
                    acc.at[pl.ds(NS * RA, TAIL)])

  pltpu.sync_copy(srcr.at[wid], src_v)
  pltpu.sync_copy(dstr.at[wid], dst_v)
  plsc.subcore_barrier()

  def chunk(j, _):
    pltpu.async_copy(p_hbm.at[src_v.at[j]], rows, sem).wait()
    pltpu.sync_copy(rows, acc.at[dst_v.at[j]], add=True)
    return 0
  lax.fori_loop(0, NCH, chunk, 0)
  plsc.subcore_barrier()
  pltpu.sync_copy(acc.at[pl.ds(s * RA, RA)], out.at[c, pl.ds(s * RA, RA)])

  @pl.when(s == NS - 1)
  def _():
    pltpu.sync_copy(acc.at[pl.ds(NS * RA, TAIL)],
                    out.at[c, pl.ds(NS * RA, TAIL)])


_edge_call = pl.kernel(
    _edge_body,
    out_type=jax.ShapeDtypeStruct((NC, NN, DH), jnp.float32),
    mesh=_MESH,
    scratch_types=[
        pltpu.VMEM((NCH, CH), jnp.int32),
        pltpu.VMEM((NCH, CH), jnp.int32),
        pltpu.VMEM((CH, DH), jnp.float32),
        pltpu.VMEM_SHARED((NN, DH), jnp.float32),
        pltpu.SemaphoreType.DMA,
    ],
)


def _dinv(d0, d1):
  return lax.rsqrt(d0[:, :1] + d1[:, :1] + 1.0)


def _mm1_body(x_ref, w_ref, d0_ref, d1_ref, o_ref):
  dinv = _dinv(d0_ref[...], d1_ref[...])
  p = lax.dot_general(x_ref[...], w_ref[...], (((1,), (0,)), ((), ())),
                      precision=lax.Precision.HIGHEST,
                      preferred_element_type=jnp.float32)
  o_ref[...] = dinv * p


def _mm_body(a0_ref, a1_ref, p_ref, b_ref, w_ref, d0_ref, d1_ref, o_ref):
  dinv = _dinv(d0_ref[...], d1_ref[...])
  h = a0_ref[...] + a1_ref[...] + p_ref[...]
  h = jnp.maximum(dinv * h + b_ref[...], 0.0)
  o_ref[...] = dinv * lax.dot_general(
      h, w_ref[...], (((1,), (0,)), ((), ())),
      precision=lax.Precision.HIGHEST, preferred_element_type=jnp.float32)


def _pool_body(a0_ref, a1_ref, p_ref, b_ref, d0_ref, d1_ref, bt_ref,
               wfc_ref, bfc_ref, emb_ref, pred_ref, ssum, cnt):
  i = pl.program_id(0)

  @pl.when(i == 0)
  def _():
    ssum[...] = jnp.zeros_like(ssum)
    cnt[...] = jnp.zeros_like(cnt)

  dinv = _dinv(d0_ref[...], d1_ref[...])
  h = a0_ref[...] + a1_ref[...] + p_ref[...]
  h = jnp.maximum(dinv * h + b_ref[...], 0.0)
  seg = jnp.reshape(bt_ref[...], (1, BR))
  ids = lax.broadcasted_iota(jnp.int32, (GG, BR), 0)
  onehot = (seg == ids).astype(jnp.float32)
  ssum[...] += lax.dot_general(onehot, h, (((1,), (0,)), ((), ())),
                               precision=lax.Precision.HIGHEST,
                               preferred_element_type=jnp.float32)
  cnt[...] += jnp.sum(onehot, axis=1, keepdims=True)

  @pl.when(i == NBLK - 1)
  def _():
    emb = ssum[...] / jnp.maximum(cnt[...], 1.0)
    emb_ref[...] = emb
    pred_ref[...] = lax.dot_general(
        emb, wfc_ref[...], (((1,), (0,)), ((), ())),
        precision=lax.Precision.HIGHEST,
        preferred_element_type=jnp.float32) + bfc_ref[...]


def _row_spec(w):
  return pl.BlockSpec((BR, w), lambda i: (i, 0))


def _full_spec(h, w):
  return pl.BlockSpec((h, w), lambda i: (0, 0))


_mm1_call = pl.pallas_call(
    _mm1_body,
    grid=(NBLK,),
    in_specs=[_row_spec(DH), _full_spec(DH, DH),
              _row_spec(DEGW), _row_spec(DEGW)],
    out_specs=_row_spec(DH),
    out_shape=jax.ShapeDtypeStruct((NN, DH), jnp.float32),
)

_mm_call = pl.pallas_call(
    _mm_body,
    grid=(NBLK,),
    in_specs=[_row_spec(DH), _row_spec(DH), _row_spec(DH),
              _full_spec(1, DH), _full_spec(DH, DH),
              _row_spec(DEGW), _row_spec(DEGW)],
    out_specs=_row_spec(DH),
    out_shape=jax.ShapeDtypeStruct((NN, DH), jnp.float32),
)

_pool_call = pl.pallas_call(
    _pool_body,
    grid=(NBLK,),
    in_specs=[_row_spec(DH), _row_spec(DH), _row_spec(DH),
              _full_spec(1, DH),
              _row_spec(DEGW), _row_spec(DEGW),
              pl.BlockSpec((1, 1, BR), lambda i: (i, 0, 0)),
              _full_spec(DH, DOUT), _full_spec(1, DOUT)],
    out_specs=[_full_spec(GG, DH), _full_spec(GG, DOUT)],
    out_shape=[jax.ShapeDtypeStruct((GG, DH), jnp.float32),
               jax.ShapeDtypeStruct((GG, DOUT), jnp.float32)],
    scratch_shapes=[pltpu.VMEM((GG, DH), jnp.float32),
                    pltpu.VMEM((GG, 1), jnp.float32)],
)


@jax.jit
def kernel(x, edge_index, batch, W1, b1, W2, b2, W3, b3, W4, b4, Wfc, bfc):
  src = edge_index[0].astype(jnp.int32).reshape(NW, NCH, CH)
  dst = edge_index[1].astype(jnp.int32).reshape(NW, NCH, CH)
  bt = batch.astype(jnp.int32).reshape(NBLK, 1, BR)
  zer = jnp.zeros((NN, DH), jnp.float32)
  zer16 = jnp.zeros((NN, DEGW), jnp.float32)

  degp = _deg_call(dst, zer16)
  d0, d1 = degp[0], degp[1]

  p = _mm1_call(x, W1, d0, d1)
  for (w, b) in ((W2, b1), (W3, b2), (W4, b3)):
    acc = _edge_call(p, src, dst, zer)
    p = _mm_call(acc[0], acc[1], p, b.reshape(1, DH), w, d0, d1)
  acc = _edge_call(p, src, dst, zer)
  emb, pred = _pool_call(acc[0], acc[1], p, b4.reshape(1, DH), d0, d1,
                         bt, Wfc, bfc.reshape(1, DOUT))
  return (emb, pred)


# speedup vs baseline: 20.3114x; 1.4290x over previous
import functools

import jax
import jax.numpy as jnp
from jax import lax
from jax.experimental import pallas as pl
from jax.experimental.pallas import tpu as pltpu
from jax.experimental.pallas import tpu_sc as plsc

NN = 10000
EE = 320000
DH = 128
DOUT = 64
GG = 64
NC = 2
NS = 16
NW = NC * NS
EPW = EE // NW
CH = 125
NCH = EPW // CH
HALF = NCH // 2
NPAIRH = HALF // 2
RA = 624
TAIL = NN - NS * RA
DEGW = 16

BR = 1000
NBLK = NN // BR

_MESH = plsc.VectorSubcoreMesh(
    core_axis_name="c", subcore_axis_name="s", num_cores=NC, num_subcores=NS)


def _deg_body(dstr, zer16, out, dst_v, buf, acc):
  c = lax.axis_index("c")
  s = lax.axis_index("s")
  wid = s * NC + c
  pltpu.sync_copy(zer16.at[pl.ds(s * RA, RA)], acc.at[pl.ds(s * RA, RA)])

  @pl.when(s == NS - 1)
  def _():
    pltpu.sync_copy(zer16.at[pl.ds(NS * RA, TAIL)],
                    acc.at[pl.ds(NS * RA, TAIL)])

  pltpu.sync_copy(dstr.at[wid], dst_v)

  def fill(i, _):
    buf[i] = jnp.ones((DEGW,), jnp.float32)
    return 0
  lax.fori_loop(0, CH, fill, 0)
  plsc.subcore_barrier()

  def chunk(j, _):
    pltpu.sync_copy(buf.at[pl.ds(0, CH)], acc.at[dst_v.at[j]], add=True)
    return 0
  lax.fori_loop(0, NCH, chunk, 0)
  plsc.subcore_barrier()
  pltpu.sync_copy(acc.at[pl.ds(s * RA, RA)], out.at[c, pl.ds(s * RA, RA)])

  @pl.when(s == NS - 1)
  def _():
    pltpu.sync_copy(acc.at[pl.ds(NS * RA, TAIL)],
                    out.at[c, pl.ds(NS * RA, TAIL)])


_deg_call = pl.kernel(
    _deg_body,
    out_type=jax.ShapeDtypeStruct((NC, NN, DEGW), jnp.float32),
    mesh=_MESH,
    scratch_types=[
        pltpu.VMEM((NCH, CH), jnp.int32),
        pltpu.VMEM((CH, DEGW), jnp.float32),
        pltpu.VMEM_SHARED((NN, DEGW), jnp.float32),
    ],
)


def _edge_body(p_hbm, srcr, dstr, zer, out, src_v, dst_v, rows0, rows1, acc,
               sem0, sem1):
  c = lax.axis_index("c")
  s = lax.axis_index("s")
  wid = s * NC + c
  pltpu.sync_copy(zer.at[pl.ds(s * RA, RA)], acc.at[pl.ds(s * RA, RA)])

  @pl.when(s == NS - 1)
  def _():
    pltpu.sync_copy(zer.at[pl.ds(NS * RA, TAIL)],
                    acc.at[pl.ds(NS * RA, TAIL)])

  plsc.subcore_barrier()

  for h in range(2):
    pltpu.sync_copy(srcr.at[wid, pl.ds(h * HALF, HALF)], src_v)
    pltpu.sync_copy(dstr.at[wid, pl.ds(h * HALF, HALF)], dst_v)
    pltpu.async_copy(p_hbm.at[src_v.at[0]], rows0, sem0)

    def pair(jj, _):
      ja = 2 * jj
      jb = 2 * jj + 1
      pltpu.make_async_copy(p_hbm.at[src_v.at[ja]], rows0, sem0).wait()
      pltpu.async_copy(p_hbm.at[src_v.at[jb]], rows1, sem1)
      pltpu.sync_copy(rows0, acc.at[dst_v.at[ja]], add=True)
      pltpu.make_async_copy(p_hbm.at[src_v.at[jb]], rows1, sem1).wait()

      @pl.when(jj < NPAIRH - 1)
      def _():
        pltpu.async_copy(p_hbm.at[src_v.at[ja + 2]], rows0, sem0)

      pltpu.sync_copy(rows1, acc.at[dst_v.at[jb]], add=True)
      return 0
    lax.fori_loop(0, NPAIRH, pair, 0)
  plsc.subcore_barrier()
  pltpu.sync_copy(acc.at[pl.ds(s * RA, RA)], out.at[c, pl.ds(s * RA, RA)])

  @pl.when(s == NS - 1)
  def _():
    pltpu.sync_copy(acc.at[pl.ds(NS * RA, TAIL)],
                    out.at[c, pl.ds(NS * RA, TAIL)])


_edge_call = pl.kernel(
    _edge_body,
    out_type=jax.ShapeDtypeStruct((NC, NN, DH), jnp.float32),
    mesh=_MESH,
    scratch_types=[
        pltpu.VMEM((HALF, CH), jnp.int32),
        pltpu.VMEM((HALF, CH), jnp.int32),
        pltpu.VMEM((CH, DH), jnp.float32),
        pltpu.VMEM((CH, DH), jnp.float32),
        pltpu.VMEM_SHARED((NN, DH), jnp.float32),
        pltpu.SemaphoreType.DMA,
        pltpu.SemaphoreType.DMA,
    ],
)


def _dinv(d0, d1):
  return lax.rsqrt(d0[:, :1] + d1[:, :1] + 1.0)


def _mm1_body(x_ref, w_ref, d0_ref, d1_ref, o_ref):
  dinv = _dinv(d0_ref[...], d1_ref[...])
  p = lax.dot_general(x_ref[...], w_ref[...], (((1,), (0,)), ((), ())),
                      precision=lax.Precision.HIGHEST,
                      preferred_element_type=jnp.float32)
  o_ref[...] = dinv * p


def _mm_body(a0_ref, a1_ref, p_ref, b_ref, w_ref, d0_ref, d1_ref, o_ref):
  dinv = _dinv(d0_ref[...], d1_ref[...])
  h = a0_ref[...] + a1_ref[...] + p_ref[...]
  h = jnp.maximum(dinv * h + b_ref[...], 0.0)
  o_ref[...] = dinv * lax.dot_general(
      h, w_ref[...], (((1,), (0,)), ((), ())),
      precision=lax.Precision.HIGHEST, preferred_element_type=jnp.float32)


def _pool_body(a0_ref, a1_ref, p_ref, b_ref, d0_ref, d1_ref, bt_ref,
               wfc_ref, bfc_ref, emb_ref, pred_ref, ssum, cnt):
  i = pl.program_id(0)

  @pl.when(i == 0)
  def _():
    ssum[...] = jnp.zeros_like(ssum)
    cnt[...] = jnp.zeros_like(cnt)

  dinv = _dinv(d0_ref[...], d1_ref[...])
  h = a0_ref[...] + a1_ref[...] + p_ref[...]
  h = jnp.maximum(dinv * h + b_ref[...], 0.0)
  seg = jnp.reshape(bt_ref[...], (1, BR))
  ids = lax.broadcasted_iota(jnp.int32, (GG, BR), 0)
  onehot = (seg == ids).astype(jnp.float32)
  ssum[...] += lax.dot_general(onehot, h, (((1,), (0,)), ((), ())),
                               precision=lax.Precision.HIGHEST,
                               preferred_element_type=jnp.float32)
  cnt[...] += jnp.sum(onehot, axis=1, keepdims=True)

  @pl.when(i == NBLK - 1)
  def _():
    emb = ssum[...] / jnp.maximum(cnt[...], 1.0)
    emb_ref[...] = emb
    pred_ref[...] = lax.dot_general(
        emb, wfc_ref[...], (((1,), (0,)), ((), ())),
        precision=lax.Precision.HIGHEST,
        preferred_element_type=jnp.float32) + bfc_ref[...]


def _row_spec(w):
  return pl.BlockSpec((BR, w), lambda i: (i, 0))


def _full_spec(h, w):
  return pl.BlockSpec((h, w), lambda i: (0, 0))


_mm1_call = pl.pallas_call(
    _mm1_body,
    grid=(NBLK,),
    in_specs=[_row_spec(DH), _full_spec(DH, DH),
              _row_spec(DEGW), _row_spec(DEGW)],
    out_specs=_row_spec(DH),
    out_shape=jax.ShapeDtypeStruct((NN, DH), jnp.float32),
)

_mm_call = pl.pallas_call(
    _mm_body,
    grid=(NBLK,),
    in_specs=[_row_spec(DH), _row_spec(DH), _row_spec(DH),
              _full_spec(1, DH), _full_spec(DH, DH),
              _row_spec(DEGW), _row_spec(DEGW)],
    out_specs=_row_spec(DH),
    out_shape=jax.ShapeDtypeStruct((NN, DH), jnp.float32),
)

_pool_call = pl.pallas_call(
    _pool_body,
    grid=(NBLK,),
    in_specs=[_row_spec(DH), _row_spec(DH), _row_spec(DH),
              _full_spec(1, DH),
              _row_spec(DEGW), _row_spec(DEGW),
              pl.BlockSpec((1, 1, BR), lambda i: (i, 0, 0)),
              _full_spec(DH, DOUT), _full_spec(1, DOUT)],
    out_specs=[_full_spec(GG, DH), _full_spec(GG, DOUT)],
    out_shape=[jax.ShapeDtypeStruct((GG, DH), jnp.float32),
               jax.ShapeDtypeStruct((GG, DOUT), jnp.float32)],
    scratch_shapes=[pltpu.VMEM((GG, DH), jnp.float32),
                    pltpu.VMEM((GG, 1), jnp.float32)],
)


@jax.jit
def kernel(x, edge_index, batch, W1, b1, W2, b2, W3, b3, W4, b4, Wfc, bfc):
  src = edge_index[0].astype(jnp.int32).reshape(NW, NCH, CH)
  dst = edge_index[1].astype(jnp.int32).reshape(NW, NCH, CH)
  bt = batch.astype(jnp.int32).reshape(NBLK, 1, BR)
  zer = jnp.zeros((NN, DH), jnp.float32)
  zer16 = jnp.zeros((NN, DEGW), jnp.float32)

  degp = _deg_call(dst, zer16)
  d0, d1 = degp[0], degp[1]

  p = _mm1_call(x, W1, d0, d1)
  for (w, b) in ((W2, b1), (W3, b2), (W4, b3)):
    acc = _edge_call(p, src, dst, zer)
    p = _mm_call(acc[0], acc[1], p, b.reshape(1, DH), w, d0, d1)
  acc = _edge_call(p, src, dst, zer)
  emb, pred = _pool_call(acc[0], acc[1], p, b4.reshape(1, DH), d0, d1,
                         bt, Wfc, bfc.reshape(1, DOUT))
  return (emb, pred)


# speedup vs baseline: 20.4438x; 1.0065x over previous
import functools

import jax
import jax.numpy as jnp
from jax import lax
from jax.experimental import pallas as pl
from jax.experimental.pallas import tpu as pltpu
from jax.experimental.pallas import tpu_sc as plsc

NN = 10000
EE = 320000
DH = 128
DOUT = 64
GG = 64
NC = 2
NS = 16
NW = NC * NS
EPW = EE // NW
CH = 125
NCH = EPW // CH
HALF = NCH // 2
NPAIRH = HALF // 2
RA = 624
TAIL = NN - NS * RA
DEGW = 16

BR = 1000
NBLK = NN // BR

_MESH = plsc.VectorSubcoreMesh(
    core_axis_name="c", subcore_axis_name="s", num_cores=NC, num_subcores=NS)


def _deg_body(dstr, zer16, out, dst_v, buf, acc):
  c = lax.axis_index("c")
  s = lax.axis_index("s")
  wid = s * NC + c
  pltpu.sync_copy(zer16.at[pl.ds(s * RA, RA)], acc.at[pl.ds(s * RA, RA)])

  @pl.when(s == NS - 1)
  def _():
    pltpu.sync_copy(zer16.at[pl.ds(NS * RA, TAIL)],
                    acc.at[pl.ds(NS * RA, TAIL)])

  pltpu.sync_copy(dstr.at[wid], dst_v)

  def fill(i, _):
    buf[i] = jnp.ones((DEGW,), jnp.float32)
    return 0
  lax.fori_loop(0, CH, fill, 0)
  plsc.subcore_barrier()

  def chunk(j, _):
    pltpu.sync_copy(buf.at[pl.ds(0, CH)], acc.at[dst_v.at[j]], add=True)
    return 0
  lax.fori_loop(0, NCH, chunk, 0)
  plsc.subcore_barrier()
  pltpu.sync_copy(acc.at[pl.ds(s * RA, RA)], out.at[c, pl.ds(s * RA, RA)])

  @pl.when(s == NS - 1)
  def _():
    pltpu.sync_copy(acc.at[pl.ds(NS * RA, TAIL)],
                    out.at[c, pl.ds(NS * RA, TAIL)])


_deg_call = pl.kernel(
    _deg_body,
    out_type=jax.ShapeDtypeStruct((NC, NN, DEGW), jnp.float32),
    mesh=_MESH,
    scratch_types=[
        pltpu.VMEM((NCH, CH), jnp.int32),
        pltpu.VMEM((CH, DEGW), jnp.float32),
        pltpu.VMEM_SHARED((NN, DEGW), jnp.float32),
    ],
)


def _edge_body(p_hbm, srcr, dstr, zer, out, src_v, dst_v, rows0, rows1, acc,
               sem0, sem1):
  c = lax.axis_index("c")
  s = lax.axis_index("s")
  wid = s * NC + c

  @pl.when(c == 0)
  def _():
    pltpu.sync_copy(p_hbm.at[pl.ds(s * RA, RA)], acc.at[pl.ds(s * RA, RA)])

  @pl.when(c != 0)
  def _():
    pltpu.sync_copy(zer.at[pl.ds(s * RA, RA)], acc.at[pl.ds(s * RA, RA)])

  @pl.when(jnp.logical_and(c == 0, s == NS - 1))
  def _():
    pltpu.sync_copy(p_hbm.at[pl.ds(NS * RA, TAIL)],
                    acc.at[pl.ds(NS * RA, TAIL)])

  @pl.when(jnp.logical_and(c != 0, s == NS - 1))
  def _():
    pltpu.sync_copy(zer.at[pl.ds(NS * RA, TAIL)],
                    acc.at[pl.ds(NS * RA, TAIL)])

  plsc.subcore_barrier()

  for h in range(2):
    pltpu.sync_copy(srcr.at[wid, pl.ds(h * HALF, HALF)], src_v)
    pltpu.sync_copy(dstr.at[wid, pl.ds(h * HALF, HALF)], dst_v)
    pltpu.async_copy(p_hbm.at[src_v.at[0]], rows0, sem0)

    def pair(jj, _):
      ja = 2 * jj
      jb = 2 * jj + 1
      pltpu.make_async_copy(p_hbm.at[src_v.at[ja]], rows0, sem0).wait()
      pltpu.async_copy(p_hbm.at[src_v.at[jb]], rows1, sem1)
      pltpu.sync_copy(rows0, acc.at[dst_v.at[ja]], add=True)
      pltpu.make_async_copy(p_hbm.at[src_v.at[jb]], rows1, sem1).wait()

      @pl.when(jj < NPAIRH - 1)
      def _():
        pltpu.async_copy(p_hbm.at[src_v.at[ja + 2]], rows0, sem0)

      pltpu.sync_copy(rows1, acc.at[dst_v.at[jb]], add=True)
      return 0
    lax.fori_loop(0, NPAIRH, pair, 0)
  plsc.subcore_barrier()
  pltpu.sync_copy(acc.at[pl.ds(s * RA, RA)], out.at[c, pl.ds(s * RA, RA)])

  @pl.when(s == NS - 1)
  def _():
    pltpu.sync_copy(acc.at[pl.ds(NS * RA, TAIL)],
                    out.at[c, pl.ds(NS * RA, TAIL)])


_edge_call = pl.kernel(
    _edge_body,
    out_type=jax.ShapeDtypeStruct((NC, NN, DH), jnp.float32),
    mesh=_MESH,
    scratch_types=[
        pltpu.VMEM((HALF, CH), jnp.int32),
        pltpu.VMEM((HALF, CH), jnp.int32),
        pltpu.VMEM((CH, DH), jnp.float32),
        pltpu.VMEM((CH, DH), jnp.float32),
        pltpu.VMEM_SHARED((NN, DH), jnp.float32),
        pltpu.SemaphoreType.DMA,
        pltpu.SemaphoreType.DMA,
    ],
)


def _dinv(d0, d1):
  return lax.rsqrt(d0[:, :1] + d1[:, :1] + 1.0)


def _mm1_body(x_ref, w_ref, d0_ref, d1_ref, o_ref):
  dinv = _dinv(d0_ref[...], d1_ref[...])
  p = lax.dot_general(x_ref[...], w_ref[...], (((1,), (0,)), ((), ())),
                      precision=lax.Precision.HIGHEST,
                      preferred_element_type=jnp.float32)
  o_ref[...] = dinv * p


def _mm_body(a0_ref, a1_ref, b_ref, w_ref, d0_ref, d1_ref, o_ref):
  dinv = _dinv(d0_ref[...], d1_ref[...])
  h = a0_ref[...] + a1_ref[...]
  h = jnp.maximum(dinv * h + b_ref[...], 0.0)
  o_ref[...] = dinv * lax.dot_general(
      h, w_ref[...], (((1,), (0,)), ((), ())),
      precision=lax.Precision.HIGHEST, preferred_element_type=jnp.float32)


def _pool_body(a0_ref, a1_ref, b_ref, d0_ref, d1_ref, bt_ref,
               wfc_ref, bfc_ref, emb_ref, pred_ref, ssum, cnt):
  i = pl.program_id(0)

  @pl.when(i == 0)
  def _():
    ssum[...] = jnp.zeros_like(ssum)
    cnt[...] = jnp.zeros_like(cnt)

  dinv = _dinv(d0_ref[...], d1_ref[...])
  h = a0_ref[...] + a1_ref[...]
  h = jnp.maximum(dinv * h + b_ref[...], 0.0)
  seg = jnp.reshape(bt_ref[...], (1, BR))
  ids = lax.broadcasted_iota(jnp.int32, (GG, BR), 0)
  onehot = (seg == ids).astype(jnp.float32)
  ssum[...] += lax.dot_general(onehot, h, (((1,), (0,)), ((), ())),
                               precision=lax.Precision.HIGHEST,
                               preferred_element_type=jnp.float32)
  cnt[...] += jnp.sum(onehot, axis=1, keepdims=True)

  @pl.when(i == NBLK - 1)
  def _():
    emb = ssum[...] / jnp.maximum(cnt[...], 1.0)
    emb_ref[...] = emb
    pred_ref[...] = lax.dot_general(
        emb, wfc_ref[...], (((1,), (0,)), ((), ())),
        precision=lax.Precision.HIGHEST,
        preferred_element_type=jnp.float32) + bfc_ref[...]


def _row_spec(w):
  return pl.BlockSpec((BR, w), lambda i: (i, 0))


def _full_spec(h, w):
  return pl.BlockSpec((h, w), lambda i: (0, 0))


_mm1_call = pl.pallas_call(
    _mm1_body,
    grid=(NBLK,),
    in_specs=[_row_spec(DH), _full_spec(DH, DH),
              _row_spec(DEGW), _row_spec(DEGW)],
    out_specs=_row_spec(DH),
    out_shape=jax.ShapeDtypeStruct((NN, DH), jnp.float32),
)

_mm_call = pl.pallas_call(
    _mm_body,
    grid=(NBLK,),
    in_specs=[_row_spec(DH), _row_spec(DH),
              _full_spec(1, DH), _full_spec(DH, DH),
              _row_spec(DEGW), _row_spec(DEGW)],
    out_specs=_row_spec(DH),
    out_shape=jax.ShapeDtypeStruct((NN, DH), jnp.float32),
)

_pool_call = pl.pallas_call(
    _pool_body,
    grid=(NBLK,),
    in_specs=[_row_spec(DH), _row_spec(DH),
              _full_spec(1, DH),
              _row_spec(DEGW), _row_spec(DEGW),
              pl.BlockSpec((1, 1, BR), lambda i: (i, 0, 0)),
              _full_spec(DH, DOUT), _full_spec(1, DOUT)],
    out_specs=[_full_spec(GG, DH), _full_spec(GG, DOUT)],
    out_shape=[jax.ShapeDtypeStruct((GG, DH), jnp.float32),
               jax.ShapeDtypeStruct((GG, DOUT), jnp.float32)],
    scratch_shapes=[pltpu.VMEM((GG, DH), jnp.float32),
                    pltpu.VMEM((GG, 1), jnp.float32)],
)


@jax.jit
def kernel(x, edge_index, batch, W1, b1, W2, b2, W3, b3, W4, b4, Wfc, bfc):
  src = edge_index[0].astype(jnp.int32).reshape(NW, NCH, CH)
  dst = edge_index[1].astype(jnp.int32).reshape(NW, NCH, CH)
  bt = batch.astype(jnp.int32).reshape(NBLK, 1, BR)
  zer = jnp.zeros((NN, DH), jnp.float32)
  zer16 = jnp.zeros((NN, DEGW), jnp.float32)

  degp = _deg_call(dst, zer16)
  d0, d1 = degp[0], degp[1]

  p = _mm1_call(x, W1, d0, d1)
  for (w, b) in ((W2, b1), (W3, b2), (W4, b3)):
    acc = _edge_call(p, src, dst, zer)
    p = _mm_call(acc[0], acc[1], b.reshape(1, DH), w, d0, d1)
  acc = _edge_call(p, src, dst, zer)
  emb, pred = _pool_call(acc[0], acc[1], b4.reshape(1, DH), d0, d1,
                         bt, Wfc, bfc.reshape(1, DOUT))
  return (emb, pred)
